# baseline (speedup 1.0000x reference)
import functools

import jax
import jax.numpy as jnp
from jax import lax
from jax.experimental import pallas as pl
from jax.experimental.pallas import tpu as pltpu
from jax.experimental.pallas import tpu_sc as plsc

BATCH = 4096
SEQ_LEN = 200
D_MODEL = 64
LANES = 16

NUM_CORES = 2
NUM_SUBCORES = 16
NUM_WORKERS = NUM_CORES * NUM_SUBCORES
SEQ_PER_WORKER = BATCH // NUM_WORKERS
NBUF = 7
LEAD = 4

_mesh = plsc.VectorSubcoreMesh(core_axis_name="c", subcore_axis_name="s")


@functools.partial(
    pl.kernel,
    mesh=_mesh,
    compiler_params=pltpu.CompilerParams(use_tc_tiling_on_sc=False),
    out_type=jax.ShapeDtypeStruct((BATCH, SEQ_LEN, D_MODEL), jnp.float32),
    scratch_types=[
        pltpu.VMEM((SEQ_PER_WORKER, SEQ_LEN), jnp.int32),
        pltpu.VMEM((NBUF, SEQ_LEN, D_MODEL), jnp.float32),
        pltpu.SemaphoreType.DMA((NBUF,)),
        pltpu.SemaphoreType.DMA((NBUF,)),
        pltpu.SemaphoreType.DMA((NBUF,)),
    ],
)
def _emb_kernel(x_hbm, tok_hbm, pos_hbm, out_hbm, idx_v, bufs, gsem, ssem, psem):
    cid = lax.axis_index("c")
    sid = lax.axis_index("s")
    wid = sid * NUM_CORES + cid
    seq_base = wid * SEQ_PER_WORKER

    pltpu.sync_copy(x_hbm.at[pl.ds(seq_base, SEQ_PER_WORKER)], idx_v)

    def start_prefill(b):
        pltpu.async_copy(pos_hbm, bufs.at[b], psem.at[b])

    def wait_prefill(b):
        pltpu.make_async_copy(pos_hbm, bufs.at[b], psem.at[b]).wait()

    def start_gather(c, b):
        wait_prefill(b)
        pltpu.async_copy(tok_hbm.at[idx_v.at[c]], bufs.at[b], gsem.at[b], add=True)

    def wait_gather(b):
        pltpu.make_async_copy(tok_hbm.at[idx_v.at[0]], bufs.at[b], gsem.at[b]).wait()

    def start_store(c, b):
        pltpu.async_copy(bufs.at[b], out_hbm.at[seq_base + c], ssem.at[b])

    def wait_store(b):
        pltpu.make_async_copy(bufs.at[b], out_hbm.at[seq_base], ssem.at[b]).wait()

    for b in range(LEAD):
        start_prefill(b)
        start_gather(b, b)
    start_prefill(LEAD % NBUF)

    def outer(go, carry):
        for b in range(NBUF):
            c = go * NBUF + b
            nslot = (b + LEAD) % NBUF

            pslot = (b + LEAD + 1) % NBUF

            @pl.when(c < SEQ_PER_WORKER - LEAD - 1)
            def _prefill():
                @pl.when(c >= NBUF - LEAD - 1)
                def _pdrain():
                    wait_store(pslot)

                start_prefill(pslot)

            @pl.when(c < SEQ_PER_WORKER - LEAD)
            def _issue():
                start_gather(c + LEAD, nslot)

            wait_gather(b)
            start_store(c, b)
        return carry

    main = SEQ_PER_WORKER - (SEQ_PER_WORKER % NBUF)
    lax.fori_loop(0, main // NBUF, outer, 0)
    for c in range(main, SEQ_PER_WORKER):
        b = c % NBUF
        wait_gather(b)
        start_store(c, b)

    for b in range(NBUF):
        wait_store(b)


@jax.jit
def kernel(x, token_table, pos_table):
    return _emb_kernel(x.astype(jnp.int32), token_table, pos_table)


# speedup vs baseline: 1.1828x; 1.1828x over previous
import functools

import jax
import jax.numpy as jnp
from jax import lax
from jax.experimental import pallas as pl
from jax.experimental.pallas import tpu as pltpu
from jax.experimental.pallas import tpu_sc as plsc

BATCH = 4096
SEQ_LEN = 200
D_MODEL = 64
LANES = 16

NUM_CORES = 2
NUM_SUBCORES = 16
NUM_WORKERS = NUM_CORES * NUM_SUBCORES
SEQ_PER_WORKER = BATCH // NUM_WORKERS
NBUF = 7
LEAD = 4

_mesh = plsc.VectorSubcoreMesh(core_axis_name="c", subcore_axis_name="s")


@functools.partial(
    pl.kernel,
    mesh=_mesh,
    compiler_params=pltpu.CompilerParams(use_tc_tiling_on_sc=False),
    out_type=jax.ShapeDtypeStruct((BATCH, SEQ_LEN, D_MODEL), jnp.float32),
    scratch_types=[
        pltpu.VMEM((SEQ_PER_WORKER, SEQ_LEN), jnp.int32),
        pltpu.VMEM((SEQ_LEN, D_MODEL), jnp.float32),
        pltpu.VMEM((NBUF, SEQ_LEN, D_MODEL), jnp.float32),
        pltpu.SemaphoreType.DMA((NBUF,)),
        pltpu.SemaphoreType.DMA((NBUF,)),
    ],
)
def _emb_kernel(x_hbm, tok_hbm, pos_hbm, out_hbm, idx_v, pos_v, bufs, gsem, ssem):
    cid = lax.axis_index("c")
    sid = lax.axis_index("s")
    wid = sid * NUM_CORES + cid
    seq_base = wid * SEQ_PER_WORKER

    pltpu.sync_copy(x_hbm.at[pl.ds(seq_base, SEQ_PER_WORKER)], idx_v)
    pltpu.sync_copy(pos_hbm, pos_v)

    def start_gather(c, b):
        pltpu.async_copy(tok_hbm.at[idx_v.at[c]], bufs.at[b], gsem.at[b])

    def wait_gather(b):
        pltpu.make_async_copy(tok_hbm.at[idx_v.at[0]], bufs.at[b], gsem.at[b]).wait()

    def start_store(c, b):
        pltpu.async_copy(bufs.at[b], out_hbm.at[seq_base + c], ssem.at[b])

    def wait_store(b):
        pltpu.make_async_copy(bufs.at[b], out_hbm.at[seq_base], ssem.at[b]).wait()

    def add_pos(b):
        buf = bufs.at[b]

        def row(i, carry):
            for k in range(D_MODEL // LANES):
                sl = pl.ds(k * LANES, LANES)
                buf[i, sl] = buf[i, sl] + pos_v[i, sl]
            return carry

        lax.fori_loop(0, SEQ_LEN, row, 0, unroll=4)

    for b in range(LEAD):
        start_gather(b, b)

    def outer(go, carry):
        for b in range(NBUF):
            c = go * NBUF + b
            nslot = (b + LEAD) % NBUF

            @pl.when(c < SEQ_PER_WORKER - LEAD)
            def _issue():
                @pl.when(c >= NBUF - LEAD)
                def _drain():
                    wait_store(nslot)

                start_gather(c + LEAD, nslot)

            wait_gather(b)
            add_pos(b)
            start_store(c, b)
        return carry

    main = SEQ_PER_WORKER - (SEQ_PER_WORKER % NBUF)
    lax.fori_loop(0, main // NBUF, outer, 0)
    for c in range(main, SEQ_PER_WORKER):
        b = c % NBUF
        wait_gather(b)
        add_pos(b)
        start_store(c, b)

    for b in range(NBUF):
        wait_store(b)


@jax.jit
def kernel(x, token_table, pos_table):
    return _emb_kernel(x.astype(jnp.int32), token_table, pos_table)
